# baseline (speedup 1.0000x reference)
import jax
import jax.numpy as jnp
from jax.experimental import pallas as pl
from jax.experimental.pallas import tpu as pltpu

TOKENS = 32768
D_MODEL = 1024
NUM_EXPERTS = 64
THRESHOLD = 0.5
TILE = 1024
NT = TOKENS // TILE


def _gate_kernel(x_ref, w_ref, b_ref, gated_ref, mask_ref):
    logits = jax.lax.dot_general(
        w_ref[...], x_ref[...],
        dimension_numbers=(((1,), (1,)), ((), ())),
        preferred_element_type=jnp.float32,
    )
    logits = logits + jnp.transpose(b_ref[...])
    probs = jax.nn.sigmoid(logits)
    mask = (probs > THRESHOLD).astype(jnp.float32)
    gated_ref[...] = probs * mask
    mask_ref[...] = mask


def kernel(x, W, b):
    b2 = b.reshape(1, NUM_EXPERTS)
    out_shape = jax.ShapeDtypeStruct((NUM_EXPERTS, TOKENS), jnp.float32)
    gated_t, mask_t = pl.pallas_call(
        _gate_kernel,
        grid=(NT,),
        in_specs=[
            pl.BlockSpec((TILE, D_MODEL), lambda i: (i, 0)),
            pl.BlockSpec((NUM_EXPERTS, D_MODEL), lambda i: (0, 0)),
            pl.BlockSpec((1, NUM_EXPERTS), lambda i: (0, 0)),
        ],
        out_specs=[
            pl.BlockSpec((NUM_EXPERTS, TILE), lambda i: (0, i)),
            pl.BlockSpec((NUM_EXPERTS, TILE), lambda i: (0, i)),
        ],
        out_shape=[out_shape, out_shape],
        compiler_params=pltpu.CompilerParams(
            dimension_semantics=("arbitrary",),
        ),
    )(x, W, b2)
    return gated_t.T, mask_t.T


# speedup vs baseline: 1.1546x; 1.1546x over previous
import jax
import jax.numpy as jnp
from jax.experimental import pallas as pl
from jax.experimental.pallas import tpu as pltpu

TOKENS = 32768
D_MODEL = 1024
NUM_EXPERTS = 64
THRESHOLD = 0.5
TILE = 2048
NT = TOKENS // TILE


def _gate_kernel(x_ref, w_ref, b_ref, gated_ref, mask_ref):
    logits = jax.lax.dot_general(
        w_ref[...], x_ref[...],
        dimension_numbers=(((1,), (1,)), ((), ())),
        preferred_element_type=jnp.float32,
    )
    logits = logits + jnp.transpose(b_ref[...])
    probs = jax.nn.sigmoid(logits)
    mask = (probs > THRESHOLD).astype(jnp.float32)
    gated_ref[...] = probs * mask
    mask_ref[...] = mask


def kernel(x, W, b):
    b2 = b.reshape(1, NUM_EXPERTS)
    out_shape = jax.ShapeDtypeStruct((NUM_EXPERTS, TOKENS), jnp.float32)
    gated_t, mask_t = pl.pallas_call(
        _gate_kernel,
        grid=(NT,),
        in_specs=[
            pl.BlockSpec((TILE, D_MODEL), lambda i: (i, 0)),
            pl.BlockSpec((NUM_EXPERTS, D_MODEL), lambda i: (0, 0)),
            pl.BlockSpec((1, NUM_EXPERTS), lambda i: (0, 0)),
        ],
        out_specs=[
            pl.BlockSpec((NUM_EXPERTS, TILE), lambda i: (0, i)),
            pl.BlockSpec((NUM_EXPERTS, TILE), lambda i: (0, i)),
        ],
        out_shape=[out_shape, out_shape],
        compiler_params=pltpu.CompilerParams(
            dimension_semantics=("arbitrary",),
        ),
    )(x, W, b2)
    return gated_t.T, mask_t.T
